# baseline (speedup 1.0000x reference)
import functools

import jax
import jax.numpy as jnp
from jax import lax
from jax.experimental import pallas as pl
from jax.experimental.pallas import tpu as pltpu
from jax.experimental.pallas import tpu_sc as plsc

SIZE = 2048
B = 1024
D = 128 * 128

NC, NS = 2, 16
NW = NC * NS
BPW = B // NW
C = 2
NCHUNK = BPW // C

_mesh = plsc.VectorSubcoreMesh(core_axis_name="c", subcore_axis_name="s")


@functools.partial(
    pl.kernel,
    out_type=(
        jax.ShapeDtypeStruct((B, D), jnp.float32),
        jax.ShapeDtypeStruct((B, D), jnp.float32),
        jax.ShapeDtypeStruct((B, 16), jnp.float32),
    ),
    mesh=_mesh,
    compiler_params=pltpu.CompilerParams(use_tc_tiling_on_sc=False),
    scratch_types=[
        pltpu.VMEM((NCHUNK, C), jnp.int32),
        pltpu.VMEM((BPW,), jnp.int32),
        pltpu.VMEM((C, D), jnp.float32),
        pltpu.VMEM((C, D), jnp.float32),
        pltpu.VMEM((BPW, 16), jnp.float32),
        pltpu.SemaphoreType.DMA,
        pltpu.SemaphoreType.DMA,
        pltpu.SemaphoreType.DMA,
    ],
)
def _gather_kernel(state_hbm, next_hbm, comb_hbm, keys2_hbm, keys_hbm,
                   out_state, out_next, out_comb,
                   idx2, idxf, buf0, buf1, cbuf, sem, ws0, ws1):
    wid = lax.axis_index("s") * NC + lax.axis_index("c")
    base = wid * BPW

    pltpu.sync_copy(keys2_hbm.at[pl.ds(wid * NCHUNK, NCHUNK)], idx2)
    pltpu.sync_copy(keys_hbm.at[pl.ds(base, BPW)], idxf)

    pltpu.async_copy(comb_hbm.at[idxf], cbuf, sem).wait()
    pltpu.sync_copy(cbuf, out_comb.at[pl.ds(base, BPW)])

    def big_phase(tab, out):
        def gather(i, buf):
            return pltpu.make_async_copy(tab.at[idx2.at[i]], buf, sem)

        def write(i, buf, ws):
            return pltpu.make_async_copy(
                buf, out.at[pl.ds(base + i * C, C)], ws)

        def body(j, carry):
            i = 2 * j

            @pl.when(j >= 1)
            def _():
                write(i - 2, buf0, ws0).wait()

            gather(i, buf0).start()
            gather(i, buf0).wait()
            write(i, buf0, ws0).start()

            @pl.when(j >= 1)
            def _():
                write(i - 1, buf1, ws1).wait()

            gather(i + 1, buf1).start()
            gather(i + 1, buf1).wait()
            write(i + 1, buf1, ws1).start()
            return carry

        lax.fori_loop(0, NCHUNK // 2, body, 0)
        write(NCHUNK - 2, buf0, ws0).wait()
        write(NCHUNK - 1, buf1, ws1).wait()

    big_phase(state_hbm, out_state)
    big_phase(next_hbm, out_next)


def kernel(state_buf, action_buf, next_state_buf, reward_buf, done_buf, keys):
    state2 = state_buf.reshape(SIZE, D)
    next2 = next_state_buf.reshape(SIZE, D)
    keys2 = keys.reshape(B // C, C)
    comb = jnp.concatenate(
        [action_buf, reward_buf, done_buf,
         jnp.zeros((SIZE, 10), jnp.float32)], axis=1)
    s, n, c = _gather_kernel(state2, next2, comb, keys2, keys)
    return (s.reshape(B, 1, 128, 128), c[:, :4],
            n.reshape(B, 1, 128, 128), c[:, 4:5], c[:, 5:6])


# speedup vs baseline: 1.0310x; 1.0310x over previous
import functools

import jax
import jax.numpy as jnp
from jax import lax
from jax.experimental import pallas as pl
from jax.experimental.pallas import tpu as pltpu
from jax.experimental.pallas import tpu_sc as plsc

SIZE = 2048
B = 1024
D = 128 * 128

NC, NS = 2, 16
NW = NC * NS
BPW = B // NW

_mesh = plsc.VectorSubcoreMesh(core_axis_name="c", subcore_axis_name="s")


@functools.partial(
    pl.kernel,
    out_type=(
        jax.ShapeDtypeStruct((B, D), jnp.float32),
        jax.ShapeDtypeStruct((B, D), jnp.float32),
        jax.ShapeDtypeStruct((B, 16), jnp.float32),
    ),
    mesh=_mesh,
    compiler_params=pltpu.CompilerParams(use_tc_tiling_on_sc=False),
    scratch_types=[
        pltpu.VMEM((BPW, 1), jnp.int32),
        pltpu.VMEM((BPW,), jnp.int32),
        pltpu.VMEM((1, D), jnp.float32),
        pltpu.VMEM((1, D), jnp.float32),
        pltpu.VMEM((1, D), jnp.float32),
        pltpu.VMEM((1, D), jnp.float32),
        pltpu.VMEM((BPW, 16), jnp.float32),
        pltpu.SemaphoreType.DMA,
        pltpu.SemaphoreType.DMA,
        pltpu.SemaphoreType.DMA,
        pltpu.SemaphoreType.DMA,
        pltpu.SemaphoreType.DMA,
        pltpu.SemaphoreType.DMA,
        pltpu.SemaphoreType.DMA,
    ],
)
def _gather_kernel(state_hbm, next_hbm, comb_hbm, keys2_hbm, keys_hbm,
                   out_state, out_next, out_comb,
                   idx2, idxf, s0, s1, n0, n1, cbuf,
                   sem, gs, gn, ws0, ws1, wn0, wn1):
    wid = lax.axis_index("s") * NC + lax.axis_index("c")
    base = wid * BPW

    pltpu.sync_copy(keys2_hbm.at[pl.ds(base, BPW)], idx2)
    pltpu.sync_copy(keys_hbm.at[pl.ds(base, BPW)], idxf)

    pltpu.async_copy(comb_hbm.at[idxf], cbuf, sem).wait()
    pltpu.sync_copy(cbuf, out_comb.at[pl.ds(base, BPW)])

    def gather(tab, k, buf, g):
        return pltpu.make_async_copy(tab.at[idx2.at[k]], buf, g)

    def write(out, k, buf, w):
        return pltpu.make_async_copy(buf, out.at[pl.ds(base + k, 1)], w)

    def body(j, carry):
        k = 2 * j

        @pl.when(j >= 1)
        def _():
            write(out_state, k - 2, s0, ws0).wait()
            write(out_next, k - 2, n0, wn0).wait()

        gather(state_hbm, k, s0, gs).start()
        gather(next_hbm, k, n0, gn).start()
        gather(state_hbm, k, s0, gs).wait()
        write(out_state, k, s0, ws0).start()
        gather(next_hbm, k, n0, gn).wait()
        write(out_next, k, n0, wn0).start()

        @pl.when(j >= 1)
        def _():
            write(out_state, k - 1, s1, ws1).wait()
            write(out_next, k - 1, n1, wn1).wait()

        gather(state_hbm, k + 1, s1, gs).start()
        gather(next_hbm, k + 1, n1, gn).start()
        gather(state_hbm, k + 1, s1, gs).wait()
        write(out_state, k + 1, s1, ws1).start()
        gather(next_hbm, k + 1, n1, gn).wait()
        write(out_next, k + 1, n1, wn1).start()
        return carry

    lax.fori_loop(0, BPW // 2, body, 0)
    write(out_state, BPW - 2, s0, ws0).wait()
    write(out_next, BPW - 2, n0, wn0).wait()
    write(out_state, BPW - 1, s1, ws1).wait()
    write(out_next, BPW - 1, n1, wn1).wait()


def kernel(state_buf, action_buf, next_state_buf, reward_buf, done_buf, keys):
    state2 = state_buf.reshape(SIZE, D)
    next2 = next_state_buf.reshape(SIZE, D)
    keys2 = keys.reshape(B, 1)
    comb = jnp.concatenate(
        [action_buf, reward_buf, done_buf,
         jnp.zeros((SIZE, 10), jnp.float32)], axis=1)
    s, n, c = _gather_kernel(state2, next2, comb, keys2, keys)
    return (s.reshape(B, 1, 128, 128), c[:, :4],
            n.reshape(B, 1, 128, 128), c[:, 4:5], c[:, 5:6])
